# baseline (device time: 76977 ns/iter reference)
import jax
import jax.numpy as jnp
from jax import lax
from jax.experimental import pallas as pl
from jax.experimental.pallas import tpu as pltpu

N = 4096
D = 1024
C = 128
NCHUNK = N // C
TAIL = NCHUNK
NSEM = NCHUNK + 1
UNROLL = 8


RB = 512
NRB = N // RB


def _body(k_ref, okeep_ref, x_ref, out_ref, out3, send_buf,
          send_sems, recv_sems):
    my_x = lax.axis_index("x")
    my_y = lax.axis_index("y")
    my_z = lax.axis_index("z")
    nbr = (my_x, 1 - my_y, my_z)

    k = k_ref[0]
    m = N - k
    base_local = jnp.where(my_y == 0, 0, k)
    base_recv = jnp.where(my_y == 0, m, 0)
    base_dst = jnp.where(my_y == 0, 0, m)

    def gather_send(i, _):
        send_buf[pl.ds(i, 1)] = x_ref[pl.ds(okeep_ref[m + i], 1)]
        return 0

    @pl.when(C <= k)
    def _():
        lax.fori_loop(0, C, gather_send, 0, unroll=UNROLL)

    barrier = pltpu.get_barrier_semaphore()
    pl.semaphore_signal(
        barrier, inc=1, device_id=nbr, device_id_type=pl.DeviceIdType.MESH
    )
    pl.semaphore_wait(barrier, 1)

    def send_rdma(src_off, dst_off, sem):
        return pltpu.make_async_remote_copy(
            src_ref=send_buf.at[pl.ds(src_off, C)],
            dst_ref=out3.at[pl.ds(dst_off, C)],
            send_sem=send_sems.at[sem],
            recv_sem=recv_sems.at[sem],
            device_id=nbr,
            device_id_type=pl.DeviceIdType.MESH,
        )

    def unreshape_block(b):
        for c in range(8):
            out_ref[b * RB:(b + 1) * RB, c * 128:(c + 1) * 128] = (
                out3[b * RB:(b + 1) * RB, c, :]
            )

    def in_keep(b):
        return jnp.where(
            my_y == 0, (b + 1) * RB <= m, b * RB >= k
        )

    for j in range(NCHUNK):

        @pl.when((j + 1) * C <= k)
        def _(j=j):
            if j > 0:
                lax.fori_loop(j * C, (j + 1) * C, gather_send, 0, unroll=UNROLL)
            send_rdma(j * C, base_dst + j * C, j).start()

    @pl.when(k % C != 0)
    def _():
        lax.fori_loop((k // C) * C, k, gather_send, 0)
        send_rdma(k - C, base_dst + k - C, TAIL).start()

    def gather_keep(i, _):
        out3[pl.ds(base_local + i, 1)] = x_ref[pl.ds(okeep_ref[i], 1)]
        return 0

    for j in range(NCHUNK):

        @pl.when((j + 1) * C <= m)
        def _(j=j):
            lax.fori_loop(j * C, (j + 1) * C, gather_keep, 0, unroll=UNROLL)

    @pl.when(m % C != 0)
    def _():
        lax.fori_loop((m // C) * C, m, gather_keep, 0)

    for b in range(NRB):

        @pl.when(in_keep(b))
        def _(b=b):
            unreshape_block(b)

    for j in range(NCHUNK):

        @pl.when((j + 1) * C <= k)
        def _(j=j):
            send_rdma(j * C, base_recv + j * C, j).wait_recv()

    @pl.when(k % C != 0)
    def _():
        send_rdma(k - C, base_recv + k - C, TAIL).wait_recv()

    for b in range(NRB):

        @pl.when(jnp.logical_not(in_keep(b)))
        def _(b=b):
            unreshape_block(b)

    for j in range(NCHUNK):

        @pl.when((j + 1) * C <= k)
        def _(j=j):
            send_rdma(j * C, base_dst + j * C, j).wait_send()

    @pl.when(k % C != 0)
    def _():
        send_rdma(k - C, base_dst + k - C, TAIL).wait_send()


def kernel(x, dest):
    my_y = lax.axis_index("y")
    send_mask = (dest != my_y).astype(jnp.int32)
    k = jnp.sum(send_mask)
    order_keep = jnp.argsort(send_mask, stable=True)

    x3 = x.astype(jnp.bfloat16).reshape(N, 8, D // 8)
    k_arr = jnp.reshape(k, (1,)).astype(jnp.int32)

    return pl.pallas_call(
        _body,
        out_shape=jax.ShapeDtypeStruct((N, D), jnp.bfloat16),
        in_specs=[
            pl.BlockSpec(memory_space=pltpu.SMEM),
            pl.BlockSpec(memory_space=pltpu.SMEM),
            pl.BlockSpec(memory_space=pltpu.VMEM),
        ],
        out_specs=pl.BlockSpec(memory_space=pltpu.VMEM),
        scratch_shapes=[
            pltpu.VMEM((N, 8, D // 8), jnp.bfloat16),
            pltpu.VMEM((N, 8, D // 8), jnp.bfloat16),
            pltpu.SemaphoreType.DMA((NSEM,)),
            pltpu.SemaphoreType.DMA((NSEM,)),
        ],
        compiler_params=pltpu.CompilerParams(collective_id=0),
    )(k_arr, order_keep.astype(jnp.int32), x3)


# device time: 72451 ns/iter; 1.0625x vs baseline; 1.0625x over previous
import jax
import jax.numpy as jnp
from jax import lax
from jax.experimental import pallas as pl
from jax.experimental.pallas import tpu as pltpu

N = 4096
D = 1024
C = 128
NCHUNK = N // C
TAIL = NCHUNK
NSEM = NCHUNK + 1
UNROLL = 8


def _body(k_ref, okeep_ref, x_ref, out_ref, send_buf,
          send_sems, recv_sems):
    my_x = lax.axis_index("x")
    my_y = lax.axis_index("y")
    my_z = lax.axis_index("z")
    nbr = (my_x, 1 - my_y, my_z)

    k = k_ref[0]
    m = N - k
    base_local = jnp.where(my_y == 0, 0, k)
    base_recv = jnp.where(my_y == 0, m, 0)
    base_dst = jnp.where(my_y == 0, 0, m)

    def gather_send(i, _):
        send_buf[pl.ds(i, 1)] = x_ref[pl.ds(okeep_ref[m + i], 1)]
        return 0

    @pl.when(C <= k)
    def _():
        lax.fori_loop(0, C, gather_send, 0, unroll=UNROLL)

    barrier = pltpu.get_barrier_semaphore()
    pl.semaphore_signal(
        barrier, inc=1, device_id=nbr, device_id_type=pl.DeviceIdType.MESH
    )
    pl.semaphore_wait(barrier, 1)

    def send_rdma(src_off, dst_off, sem):
        return pltpu.make_async_remote_copy(
            src_ref=send_buf.at[pl.ds(src_off, C)],
            dst_ref=out_ref.at[pl.ds(dst_off, C)],
            send_sem=send_sems.at[sem],
            recv_sem=recv_sems.at[sem],
            device_id=nbr,
            device_id_type=pl.DeviceIdType.MESH,
        )

    for j in range(NCHUNK):

        @pl.when((j + 1) * C <= k)
        def _(j=j):
            if j > 0:
                lax.fori_loop(j * C, (j + 1) * C, gather_send, 0, unroll=UNROLL)
            send_rdma(j * C, base_dst + j * C, j).start()

    @pl.when(k % C != 0)
    def _():
        lax.fori_loop((k // C) * C, k, gather_send, 0)
        send_rdma(k - C, base_dst + k - C, TAIL).start()

    def gather_keep(i, _):
        out_ref[pl.ds(base_local + i, 1)] = x_ref[pl.ds(okeep_ref[i], 1)]
        return 0

    for j in range(NCHUNK):

        @pl.when((j + 1) * C <= m)
        def _(j=j):
            lax.fori_loop(j * C, (j + 1) * C, gather_keep, 0, unroll=UNROLL)

    @pl.when(m % C != 0)
    def _():
        lax.fori_loop((m // C) * C, m, gather_keep, 0)

    for j in range(NCHUNK):

        @pl.when((j + 1) * C <= k)
        def _(j=j):
            send_rdma(j * C, base_recv + j * C, j).wait_recv()

    @pl.when(k % C != 0)
    def _():
        send_rdma(k - C, base_recv + k - C, TAIL).wait_recv()

    for j in range(NCHUNK):

        @pl.when((j + 1) * C <= k)
        def _(j=j):
            send_rdma(j * C, base_dst + j * C, j).wait_send()

    @pl.when(k % C != 0)
    def _():
        send_rdma(k - C, base_dst + k - C, TAIL).wait_send()


def kernel(x, dest):
    my_y = lax.axis_index("y")
    send_mask = (dest != my_y).astype(jnp.int32)
    k = jnp.sum(send_mask)
    order_keep = jnp.argsort(send_mask, stable=True)

    x3 = x.astype(jnp.bfloat16).reshape(N, 8, D // 8)
    k_arr = jnp.reshape(k, (1,)).astype(jnp.int32)

    out3 = pl.pallas_call(
        _body,
        out_shape=jax.ShapeDtypeStruct((N, 8, D // 8), jnp.bfloat16),
        in_specs=[
            pl.BlockSpec(memory_space=pltpu.SMEM),
            pl.BlockSpec(memory_space=pltpu.SMEM),
            pl.BlockSpec(memory_space=pltpu.VMEM),
        ],
        out_specs=pl.BlockSpec(memory_space=pltpu.VMEM),
        scratch_shapes=[
            pltpu.VMEM((N, 8, D // 8), jnp.bfloat16),
            pltpu.SemaphoreType.DMA((NSEM,)),
            pltpu.SemaphoreType.DMA((NSEM,)),
        ],
        compiler_params=pltpu.CompilerParams(collective_id=0),
    )(k_arr, order_keep.astype(jnp.int32), x3)
    return out3.reshape(N, D)
